# Initial kernel scaffold; baseline (speedup 1.0000x reference)
import jax
import jax.numpy as jnp
from jax.experimental import pallas as pl


def kernel(features, edge_index, W1, W2):
    raise NotImplementedError("write your pallas kernel here")


# speedup vs baseline: 3.4578x; 3.4578x over previous
import functools

import jax
import jax.numpy as jnp
from jax import lax
from jax.experimental import pallas as pl
from jax.experimental.pallas import tpu as pltpu
from jax.experimental.pallas import tpu_sc as plsc

N = 10000
D = 128
E = 320000

NC = 2
NS = 16
L = 16
NW = NC * NS

K = 128
CH = 80
TE = CH * K
EP = NW * TE
EROWS = EP // K

NP = 10240
RPT = NP // NS
DROWS = EROWS // NS

BN = 512
GB = NP // BN

_mesh = plsc.VectorSubcoreMesh(core_axis_name="c", subcore_axis_name="s")


def _zero_vmem_2d(ref, nrows, qcols):
    def body(i, carry):
        for q in range(qcols):
            ref[i, pl.ds(q * L, L)] = jnp.zeros((L,), jnp.float32)
        return carry
    lax.fori_loop(0, nrows, body, 0)


def _zero_vmem_1d(ref, nelems):
    def body(i, carry):
        ref[pl.ds(i * L, L)] = jnp.zeros((L,), jnp.float32)
        return carry
    lax.fori_loop(0, nelems // L, body, 0)


HR = NP // K
TROWS = EROWS // NW


@functools.partial(
    pl.kernel,
    out_type=jax.ShapeDtypeStruct((NC, 2 * HR, K), jnp.int32),
    mesh=_mesh,
    scratch_types=[
        pltpu.VMEM((TROWS, K), jnp.int32),
        pltpu.VMEM((NP,), jnp.int32),
        pltpu.VMEM((HR, K), jnp.int32),
        pltpu.VMEM((HR,), jnp.int32),
        pltpu.VMEM((HR,), jnp.int32),
        pltpu.VMEM_SHARED((2 * HR, K), jnp.int32),
    ],
    compiler_params=pltpu.CompilerParams(needs_layout_passes=False),
)
def _deg_kernel(srci, dsti, out, idxv, hist, stage, idro, idri, acc):
    c = lax.axis_index("c")
    s = lax.axis_index("s")
    wid = s * NC + c

    def zs(i, carry):
        for q in range(K // L):
            stage[i, pl.ds(q * L, L)] = jnp.zeros((L,), jnp.int32)
        return carry
    lax.fori_loop(0, HR, zs, 0)

    def zi(i, carry):
        idro[pl.ds(i * L, L)] = lax.iota(jnp.int32, L) + i * L
        idri[pl.ds(i * L, L)] = lax.iota(jnp.int32, L) + (HR + i * L)
        return carry
    lax.fori_loop(0, HR // L, zi, 0)

    zr = (2 * HR) // NS
    pltpu.sync_copy(stage.at[pl.ds(0, zr)], acc.at[pl.ds(s * zr, zr)])
    plsc.subcore_barrier()

    for idx_hbm, idr in ((srci, idro), (dsti, idri)):
        def zh(i, carry):
            hist[pl.ds(i * L, L)] = jnp.zeros((L,), jnp.int32)
            return carry
        lax.fori_loop(0, NP // L, zh, 0)
        pltpu.sync_copy(idx_hbm.at[pl.ds(wid * TROWS, TROWS)], idxv)

        def body(i, carry):
            r = i // (K // L)
            q = i % (K // L)
            vec = idxv[r, pl.ds(q * L, L)]
            cnt, last = plsc.scan_count(vec)
            vals = plsc.load_gather(hist, [vec], mask=last)
            plsc.store_scatter(hist, [vec], vals + cnt, mask=last)
            return carry
        lax.fori_loop(0, TROWS * (K // L), body, 0)

        def pack(r, carry):
            for q in range(K // L):
                stage[r, pl.ds(q * L, L)] = hist[pl.ds(r * K + q * L, L)]
            return carry
        lax.fori_loop(0, HR, pack, 0)
        pltpu.sync_copy(stage, acc.at[idr], add=True)
    plsc.subcore_barrier()

    @pl.when(s < 10)
    def _():
        pltpu.sync_copy(acc.at[pl.ds(s * 16, 16)],
                        out.at[c, pl.ds(s * 16, 16)])


DH = 64


@functools.partial(
    pl.kernel,
    out_type=jax.ShapeDtypeStruct((NC, NP, DH), jnp.float32),
    mesh=_mesh,
    scratch_types=[
        pltpu.VMEM((CH, K), jnp.int32),
        pltpu.VMEM((CH, K), jnp.int32),
        pltpu.VMEM((K, DH), jnp.float32),
        pltpu.VMEM((K, DH), jnp.float32),
        pltpu.SemaphoreType.DMA,
        pltpu.SemaphoreType.DMA,
        pltpu.VMEM_SHARED((NP, DH), jnp.float32),
    ],
    compiler_params=pltpu.CompilerParams(use_tc_tiling_on_sc=False),
)
def _mp_kernel(h, srci, dsti, out, idxs, idxd, rows0, rows1, sem0, sem1, acc):
    c = lax.axis_index("c")
    s = lax.axis_index("s")
    wid = s * NC + c

    pltpu.sync_copy(srci.at[pl.ds(wid * CH, CH)], idxs)
    pltpu.sync_copy(dsti.at[pl.ds(wid * CH, CH)], idxd)

    _zero_vmem_2d(rows0, K, DH // L)
    for t in range(RPT // K):
        pltpu.sync_copy(rows0, acc.at[pl.ds(s * RPT + t * K, K)])
    plsc.subcore_barrier()

    pltpu.async_copy(h.at[idxs.at[0]], rows0, sem0)
    pltpu.async_copy(h.at[idxs.at[1]], rows1, sem1)

    def outer(i, carry):
        g0 = i * 2
        for b, (buf, sem) in enumerate(((rows0, sem0), (rows1, sem1))):
            g = g0 + b
            pltpu.make_async_copy(h.at[idxs.at[g]], buf, sem).wait()
            pltpu.sync_copy(buf, acc.at[idxd.at[g]], add=True)
            nxt = jnp.minimum(g + 2, CH - 1)
            pltpu.async_copy(h.at[idxs.at[nxt]], buf, sem)
        return carry
    lax.fori_loop(0, CH // 2, outer, 0)
    pltpu.make_async_copy(h.at[idxs.at[CH - 1]], rows0, sem0).wait()
    pltpu.make_async_copy(h.at[idxs.at[CH - 1]], rows1, sem1).wait()

    plsc.subcore_barrier()
    for t in range(RPT // K):
        pltpu.sync_copy(acc.at[pl.ds(s * RPT + t * K, K)],
                        out.at[c, pl.ds(s * RPT + t * K, K)])


def _norm(deg):
    return lax.rsqrt(jnp.maximum(deg, 1.0))


def _scale_body(x_ref, do0_ref, do1_ref, oa_ref, ob_ref):
    scaled = x_ref[...] * _norm(do0_ref[...] + do1_ref[...])
    oa_ref[...] = scaled[:, :DH]
    ob_ref[...] = scaled[:, DH:]


def _mid_body(pa0_ref, pa1_ref, pb0_ref, pb1_ref, di0_ref, di1_ref,
              do0_ref, do1_ref, w_ref, oa_ref, ob_ref):
    agg = jnp.concatenate(
        [pa0_ref[...] + pa1_ref[...], pb0_ref[...] + pb1_ref[...]], axis=1)
    agg = agg * _norm(di0_ref[...] + di1_ref[...])
    hidden = jnp.dot(agg, w_ref[...], preferred_element_type=jnp.float32)
    scaled = jnp.maximum(hidden, 0.0) * _norm(do0_ref[...] + do1_ref[...])
    oa_ref[...] = scaled[:, :DH]
    ob_ref[...] = scaled[:, DH:]


def _final_body(pa0_ref, pa1_ref, pb0_ref, pb1_ref, di0_ref, di1_ref,
                w_ref, o_ref):
    agg = jnp.concatenate(
        [pa0_ref[...] + pa1_ref[...], pb0_ref[...] + pb1_ref[...]], axis=1)
    agg = agg * _norm(di0_ref[...] + di1_ref[...])
    o_ref[...] = jnp.dot(agg, w_ref[...], preferred_element_type=jnp.float32)


_row_spec = pl.BlockSpec((BN, D), lambda i: (i, 0))
_half_spec = pl.BlockSpec((BN, DH), lambda i: (i, 0))
_col_spec = pl.BlockSpec((BN, 1), lambda i: (i, 0))
_w_spec = pl.BlockSpec((D, D), lambda i: (0, 0))
_out_struct = jax.ShapeDtypeStruct((NP, D), jnp.float32)
_half_struct = jax.ShapeDtypeStruct((NP, DH), jnp.float32)

_scale_call = pl.pallas_call(
    _scale_body, grid=(GB,),
    in_specs=[_row_spec, _col_spec, _col_spec],
    out_specs=(_half_spec, _half_spec),
    out_shape=(_half_struct, _half_struct))

_mid_call = pl.pallas_call(
    _mid_body, grid=(GB,),
    in_specs=[_half_spec, _half_spec, _half_spec, _half_spec,
              _col_spec, _col_spec, _col_spec, _col_spec, _w_spec],
    out_specs=(_half_spec, _half_spec),
    out_shape=(_half_struct, _half_struct))

_final_call = pl.pallas_call(
    _final_body, grid=(GB,),
    in_specs=[_half_spec, _half_spec, _half_spec, _half_spec,
              _col_spec, _col_spec, _w_spec],
    out_specs=_row_spec, out_shape=_out_struct)


def kernel(features, edge_index, W1, W2):
    src = edge_index[0].astype(jnp.int32)
    dst = edge_index[1].astype(jnp.int32)
    pad = jnp.full((EP - E,), N, jnp.int32)
    srcp = jnp.concatenate([src, pad]).reshape(EROWS, K)
    dstp = jnp.concatenate([dst, pad]).reshape(EROWS, K)
    xpad = jnp.pad(features, ((0, NP - N), (0, 0)))

    deg = _deg_kernel(srcp, dstp)
    degf = deg.reshape(NC, 2, NP).astype(jnp.float32)
    do0 = degf[0, 0].reshape(NP, 1)
    do1 = degf[1, 0].reshape(NP, 1)
    di0 = degf[0, 1].reshape(NP, 1)
    di1 = degf[1, 1].reshape(NP, 1)

    ha, hb = _scale_call(xpad, do0, do1)
    pa = _mp_kernel(ha, srcp, dstp)
    pb = _mp_kernel(hb, srcp, dstp)
    ha2, hb2 = _mid_call(pa[0], pa[1], pb[0], pb[1],
                         di0, di1, do0, do1, W1)
    pa2 = _mp_kernel(ha2, srcp, dstp)
    pb2 = _mp_kernel(hb2, srcp, dstp)
    out = _final_call(pa2[0], pa2[1], pb2[0], pb2[1], di0, di1, W2)
    return out[:N]


# speedup vs baseline: 3.4883x; 1.0088x over previous
import functools

import jax
import jax.numpy as jnp
from jax import lax
from jax.experimental import pallas as pl
from jax.experimental.pallas import tpu as pltpu
from jax.experimental.pallas import tpu_sc as plsc

N = 10000
D = 128
E = 320000

NC = 2
NS = 16
L = 16
NW = NC * NS

K = 128
CH = 80
TE = CH * K
EP = NW * TE
EROWS = EP // K

NP = 10240
RPT = NP // NS
DROWS = EROWS // NS

BN = 512
GB = NP // BN

_mesh = plsc.VectorSubcoreMesh(core_axis_name="c", subcore_axis_name="s")


def _zero_vmem_2d(ref, nrows, qcols):
    def body(i, carry):
        for q in range(qcols):
            ref[i, pl.ds(q * L, L)] = jnp.zeros((L,), jnp.float32)
        return carry
    lax.fori_loop(0, nrows, body, 0)


def _zero_vmem_1d(ref, nelems):
    def body(i, carry):
        ref[pl.ds(i * L, L)] = jnp.zeros((L,), jnp.float32)
        return carry
    lax.fori_loop(0, nelems // L, body, 0)


HR = NP // K
TROWS = EROWS // NW


@functools.partial(
    pl.kernel,
    out_type=jax.ShapeDtypeStruct((NC, 2 * HR, K), jnp.int32),
    mesh=_mesh,
    scratch_types=[
        pltpu.VMEM((TROWS, K), jnp.int32),
        pltpu.VMEM((NP,), jnp.int32),
        pltpu.VMEM((HR, K), jnp.int32),
        pltpu.VMEM((HR,), jnp.int32),
        pltpu.VMEM((HR,), jnp.int32),
        pltpu.VMEM_SHARED((2 * HR, K), jnp.int32),
    ],
    compiler_params=pltpu.CompilerParams(needs_layout_passes=False),
)
def _deg_kernel(srci, dsti, out, idxv, hist, stage, idro, idri, acc):
    c = lax.axis_index("c")
    s = lax.axis_index("s")
    wid = s * NC + c

    def zs(i, carry):
        for q in range(K // L):
            stage[i, pl.ds(q * L, L)] = jnp.zeros((L,), jnp.int32)
        return carry
    lax.fori_loop(0, HR, zs, 0)

    def zi(i, carry):
        idro[pl.ds(i * L, L)] = lax.iota(jnp.int32, L) + i * L
        idri[pl.ds(i * L, L)] = lax.iota(jnp.int32, L) + (HR + i * L)
        return carry
    lax.fori_loop(0, HR // L, zi, 0)

    zr = (2 * HR) // NS
    pltpu.sync_copy(stage.at[pl.ds(0, zr)], acc.at[pl.ds(s * zr, zr)])
    plsc.subcore_barrier()

    for idx_hbm, idr in ((srci, idro), (dsti, idri)):
        def zh(i, carry):
            hist[pl.ds(i * L, L)] = jnp.zeros((L,), jnp.int32)
            return carry
        lax.fori_loop(0, NP // L, zh, 0)
        pltpu.sync_copy(idx_hbm.at[pl.ds(wid * TROWS, TROWS)], idxv)

        def body(i, carry):
            r = i // (K // L)
            q = i % (K // L)
            vec = idxv[r, pl.ds(q * L, L)]
            cnt, last = plsc.scan_count(vec)
            vals = plsc.load_gather(hist, [vec], mask=last)
            plsc.store_scatter(hist, [vec], vals + cnt, mask=last)
            return carry
        lax.fori_loop(0, TROWS * (K // L), body, 0)

        def pack(r, carry):
            for q in range(K // L):
                stage[r, pl.ds(q * L, L)] = hist[pl.ds(r * K + q * L, L)]
            return carry
        lax.fori_loop(0, HR, pack, 0)
        pltpu.sync_copy(stage, acc.at[idr], add=True)
    plsc.subcore_barrier()

    @pl.when(s < 10)
    def _():
        pltpu.sync_copy(acc.at[pl.ds(s * 16, 16)],
                        out.at[c, pl.ds(s * 16, 16)])


DH = 64


@functools.partial(
    pl.kernel,
    out_type=jax.ShapeDtypeStruct((NC, NP, DH), jnp.float32),
    mesh=_mesh,
    scratch_types=[
        pltpu.VMEM((CH, K), jnp.int32),
        pltpu.VMEM((CH, K), jnp.int32),
        [pltpu.VMEM((K, DH), jnp.float32) for _ in range(4)],
        [pltpu.SemaphoreType.DMA for _ in range(4)],
        [pltpu.SemaphoreType.DMA for _ in range(4)],
        pltpu.VMEM_SHARED((NP, DH), jnp.float32),
    ],
    compiler_params=pltpu.CompilerParams(use_tc_tiling_on_sc=False),
)
def _mp_kernel(h, srci, dsti, out, idxs, idxd, bufs, gsems, ssems, acc):
    c = lax.axis_index("c")
    s = lax.axis_index("s")
    wid = s * NC + c

    pltpu.sync_copy(srci.at[pl.ds(wid * CH, CH)], idxs)
    pltpu.sync_copy(dsti.at[pl.ds(wid * CH, CH)], idxd)

    _zero_vmem_2d(bufs[0], K, DH // L)
    for t in range(RPT // K):
        pltpu.sync_copy(bufs[0], acc.at[pl.ds(s * RPT + t * K, K)])
    plsc.subcore_barrier()

    pltpu.async_copy(h.at[idxs.at[0]], bufs[0], gsems[0])
    pltpu.async_copy(h.at[idxs.at[1]], bufs[1], gsems[1])

    def outer(i, carry):
        g0 = i * 4
        for b in range(4):
            g = g0 + b
            bp = (b + 2) % 4
            if b < 2:
                @pl.when(i > 0)
                def _():
                    pltpu.make_async_copy(
                        bufs[bp], acc.at[idxd.at[g - 2]], ssems[bp]).wait()
            else:
                pltpu.make_async_copy(
                    bufs[bp], acc.at[idxd.at[g - 2]], ssems[bp]).wait()
            nxt = jnp.minimum(g + 2, CH - 1)
            pltpu.async_copy(h.at[idxs.at[nxt]], bufs[bp], gsems[bp])
            pltpu.make_async_copy(h.at[idxs.at[g]], bufs[b], gsems[b]).wait()
            pltpu.async_copy(bufs[b], acc.at[idxd.at[g]], ssems[b], add=True)
        return carry
    lax.fori_loop(0, CH // 4, outer, 0)
    pltpu.make_async_copy(h.at[idxs.at[CH - 1]], bufs[0], gsems[0]).wait()
    pltpu.make_async_copy(h.at[idxs.at[CH - 1]], bufs[1], gsems[1]).wait()
    pltpu.make_async_copy(bufs[2], acc.at[idxd.at[CH - 2]], ssems[2]).wait()
    pltpu.make_async_copy(bufs[3], acc.at[idxd.at[CH - 1]], ssems[3]).wait()

    plsc.subcore_barrier()
    for t in range(RPT // K):
        pltpu.sync_copy(acc.at[pl.ds(s * RPT + t * K, K)],
                        out.at[c, pl.ds(s * RPT + t * K, K)])


def _norm(deg):
    return lax.rsqrt(jnp.maximum(deg, 1.0))


def _scale_body(x_ref, do0_ref, do1_ref, oa_ref, ob_ref):
    scaled = x_ref[...] * _norm(do0_ref[...] + do1_ref[...])
    oa_ref[...] = scaled[:, :DH]
    ob_ref[...] = scaled[:, DH:]


def _mid_body(pa0_ref, pa1_ref, pb0_ref, pb1_ref, di0_ref, di1_ref,
              do0_ref, do1_ref, w_ref, oa_ref, ob_ref):
    agg = jnp.concatenate(
        [pa0_ref[...] + pa1_ref[...], pb0_ref[...] + pb1_ref[...]], axis=1)
    agg = agg * _norm(di0_ref[...] + di1_ref[...])
    hidden = jnp.dot(agg, w_ref[...], preferred_element_type=jnp.float32)
    scaled = jnp.maximum(hidden, 0.0) * _norm(do0_ref[...] + do1_ref[...])
    oa_ref[...] = scaled[:, :DH]
    ob_ref[...] = scaled[:, DH:]


def _final_body(pa0_ref, pa1_ref, pb0_ref, pb1_ref, di0_ref, di1_ref,
                w_ref, o_ref):
    agg = jnp.concatenate(
        [pa0_ref[...] + pa1_ref[...], pb0_ref[...] + pb1_ref[...]], axis=1)
    agg = agg * _norm(di0_ref[...] + di1_ref[...])
    o_ref[...] = jnp.dot(agg, w_ref[...], preferred_element_type=jnp.float32)


_row_spec = pl.BlockSpec((BN, D), lambda i: (i, 0))
_half_spec = pl.BlockSpec((BN, DH), lambda i: (i, 0))
_col_spec = pl.BlockSpec((BN, 1), lambda i: (i, 0))
_w_spec = pl.BlockSpec((D, D), lambda i: (0, 0))
_out_struct = jax.ShapeDtypeStruct((NP, D), jnp.float32)
_half_struct = jax.ShapeDtypeStruct((NP, DH), jnp.float32)

_scale_call = pl.pallas_call(
    _scale_body, grid=(GB,),
    in_specs=[_row_spec, _col_spec, _col_spec],
    out_specs=(_half_spec, _half_spec),
    out_shape=(_half_struct, _half_struct))

_mid_call = pl.pallas_call(
    _mid_body, grid=(GB,),
    in_specs=[_half_spec, _half_spec, _half_spec, _half_spec,
              _col_spec, _col_spec, _col_spec, _col_spec, _w_spec],
    out_specs=(_half_spec, _half_spec),
    out_shape=(_half_struct, _half_struct))

_final_call = pl.pallas_call(
    _final_body, grid=(GB,),
    in_specs=[_half_spec, _half_spec, _half_spec, _half_spec,
              _col_spec, _col_spec, _w_spec],
    out_specs=_row_spec, out_shape=_out_struct)


def kernel(features, edge_index, W1, W2):
    src = edge_index[0].astype(jnp.int32)
    dst = edge_index[1].astype(jnp.int32)
    pad = jnp.full((EP - E,), N, jnp.int32)
    srcp = jnp.concatenate([src, pad]).reshape(EROWS, K)
    dstp = jnp.concatenate([dst, pad]).reshape(EROWS, K)
    xpad = jnp.pad(features, ((0, NP - N), (0, 0)))

    deg = _deg_kernel(srcp, dstp)
    degf = deg.reshape(NC, 2, NP).astype(jnp.float32)
    do0 = degf[0, 0].reshape(NP, 1)
    do1 = degf[1, 0].reshape(NP, 1)
    di0 = degf[0, 1].reshape(NP, 1)
    di1 = degf[1, 1].reshape(NP, 1)

    ha, hb = _scale_call(xpad, do0, do1)
    pa = _mp_kernel(ha, srcp, dstp)
    pb = _mp_kernel(hb, srcp, dstp)
    ha2, hb2 = _mid_call(pa[0], pa[1], pb[0], pb[1],
                         di0, di1, do0, do1, W1)
    pa2 = _mp_kernel(ha2, srcp, dstp)
    pb2 = _mp_kernel(hb2, srcp, dstp)
    out = _final_call(pa2[0], pa2[1], pb2[0], pb2[1], di0, di1, W2)
    return out[:N]


# speedup vs baseline: 5.4483x; 1.5619x over previous
import functools

import jax
import jax.numpy as jnp
from jax import lax
from jax.experimental import pallas as pl
from jax.experimental.pallas import tpu as pltpu
from jax.experimental.pallas import tpu_sc as plsc

N = 10000
D = 128
E = 320000

NC = 2
NS = 16
L = 16
NW = NC * NS

K = 128
CH = 80
TE = CH * K
EP = NW * TE
EROWS = EP // K

NP = 10240
RPT = NP // NS
DROWS = EROWS // NS

BN = 512
GB = NP // BN

_mesh = plsc.VectorSubcoreMesh(core_axis_name="c", subcore_axis_name="s")


def _zero_vmem_2d(ref, nrows, qcols):
    def body(i, carry):
        for q in range(qcols):
            ref[i, pl.ds(q * L, L)] = jnp.zeros((L,), jnp.float32)
        return carry
    lax.fori_loop(0, nrows, body, 0)


def _zero_vmem_1d(ref, nelems):
    def body(i, carry):
        ref[pl.ds(i * L, L)] = jnp.zeros((L,), jnp.float32)
        return carry
    lax.fori_loop(0, nelems // L, body, 0)


HR = NP // K
TROWS = EROWS // NW


@functools.partial(
    pl.kernel,
    out_type=jax.ShapeDtypeStruct((NC, 2 * HR, K), jnp.int32),
    mesh=_mesh,
    scratch_types=[
        pltpu.VMEM((TROWS, K), jnp.int32),
        pltpu.VMEM((NP,), jnp.int32),
        pltpu.VMEM((HR, K), jnp.int32),
        pltpu.VMEM((HR,), jnp.int32),
        pltpu.VMEM((HR,), jnp.int32),
        pltpu.VMEM_SHARED((2 * HR, K), jnp.int32),
    ],
    compiler_params=pltpu.CompilerParams(needs_layout_passes=False),
)
def _deg_kernel(srci, dsti, out, idxv, hist, stage, idro, idri, acc):
    c = lax.axis_index("c")
    s = lax.axis_index("s")
    wid = s * NC + c

    def zs(i, carry):
        for q in range(K // L):
            stage[i, pl.ds(q * L, L)] = jnp.zeros((L,), jnp.int32)
        return carry
    lax.fori_loop(0, HR, zs, 0)

    def zi(i, carry):
        idro[pl.ds(i * L, L)] = lax.iota(jnp.int32, L) + i * L
        idri[pl.ds(i * L, L)] = lax.iota(jnp.int32, L) + (HR + i * L)
        return carry
    lax.fori_loop(0, HR // L, zi, 0)

    zr = (2 * HR) // NS
    pltpu.sync_copy(stage.at[pl.ds(0, zr)], acc.at[pl.ds(s * zr, zr)])
    plsc.subcore_barrier()

    for idx_hbm, idr in ((srci, idro), (dsti, idri)):
        def zh(i, carry):
            hist[pl.ds(i * L, L)] = jnp.zeros((L,), jnp.int32)
            return carry
        lax.fori_loop(0, NP // L, zh, 0)
        pltpu.sync_copy(idx_hbm.at[pl.ds(wid * TROWS, TROWS)], idxv)

        def body(i, carry):
            r = i // (K // L)
            q = i % (K // L)
            vec = idxv[r, pl.ds(q * L, L)]
            cnt, last = plsc.scan_count(vec)
            vals = plsc.load_gather(hist, [vec], mask=last)
            plsc.store_scatter(hist, [vec], vals + cnt, mask=last)
            return carry
        lax.fori_loop(0, TROWS * (K // L), body, 0)

        def pack(r, carry):
            for q in range(K // L):
                stage[r, pl.ds(q * L, L)] = hist[pl.ds(r * K + q * L, L)]
            return carry
        lax.fori_loop(0, HR, pack, 0)
        pltpu.sync_copy(stage, acc.at[idr], add=True)
    plsc.subcore_barrier()

    @pl.when(s < 10)
    def _():
        pltpu.sync_copy(acc.at[pl.ds(s * 16, 16)],
                        out.at[c, pl.ds(s * 16, 16)])


DH = 64


CH2 = EROWS // NS


def _edge_sweep(h, idxs, idxd, bufs, gsems, ssems, acc):
    pltpu.async_copy(h.at[idxs.at[0]], bufs[0], gsems[0])
    pltpu.async_copy(h.at[idxs.at[1]], bufs[1], gsems[1])

    def outer(i, carry):
        g0 = i * 4
        for b in range(4):
            g = g0 + b
            bp = (b + 2) % 4
            if b < 2:
                @pl.when(i > 0)
                def _():
                    pltpu.make_async_copy(
                        bufs[bp], acc.at[idxd.at[g - 2]], ssems[bp]).wait()
            else:
                pltpu.make_async_copy(
                    bufs[bp], acc.at[idxd.at[g - 2]], ssems[bp]).wait()
            nxt = jnp.minimum(g + 2, CH2 - 1)
            pltpu.async_copy(h.at[idxs.at[nxt]], bufs[bp], gsems[bp])
            pltpu.make_async_copy(h.at[idxs.at[g]], bufs[b], gsems[b]).wait()
            pltpu.async_copy(bufs[b], acc.at[idxd.at[g]], ssems[b], add=True)
        return carry
    lax.fori_loop(0, CH2 // 4, outer, 0)
    pltpu.make_async_copy(h.at[idxs.at[CH2 - 1]], bufs[0], gsems[0]).wait()
    pltpu.make_async_copy(h.at[idxs.at[CH2 - 1]], bufs[1], gsems[1]).wait()
    pltpu.make_async_copy(bufs[2], acc.at[idxd.at[CH2 - 2]], ssems[2]).wait()
    pltpu.make_async_copy(bufs[3], acc.at[idxd.at[CH2 - 1]], ssems[3]).wait()


@functools.partial(
    pl.kernel,
    out_type=jax.ShapeDtypeStruct((NC, NP, DH), jnp.float32),
    mesh=_mesh,
    scratch_types=[
        pltpu.VMEM((CH2, K), jnp.int32),
        pltpu.VMEM((CH2, K), jnp.int32),
        [pltpu.VMEM((K, DH), jnp.float32) for _ in range(4)],
        [pltpu.SemaphoreType.DMA for _ in range(4)],
        [pltpu.SemaphoreType.DMA for _ in range(4)],
        pltpu.VMEM_SHARED((NP, DH), jnp.float32),
    ],
    compiler_params=pltpu.CompilerParams(use_tc_tiling_on_sc=False),
)
def _mp_kernel(ha, hb, srci, dsti, out, idxs, idxd, bufs, gsems, ssems, acc):
    c = lax.axis_index("c")
    s = lax.axis_index("s")

    pltpu.sync_copy(srci.at[pl.ds(s * CH2, CH2)], idxs)
    pltpu.sync_copy(dsti.at[pl.ds(s * CH2, CH2)], idxd)

    _zero_vmem_2d(bufs[0], K, DH // L)
    for t in range(RPT // K):
        pltpu.sync_copy(bufs[0], acc.at[pl.ds(s * RPT + t * K, K)])
    plsc.subcore_barrier()

    @pl.when(c == 0)
    def _():
        _edge_sweep(ha, idxs, idxd, bufs, gsems, ssems, acc)

    @pl.when(c == 1)
    def _():
        _edge_sweep(hb, idxs, idxd, bufs, gsems, ssems, acc)

    plsc.subcore_barrier()
    for t in range(RPT // K):
        pltpu.sync_copy(acc.at[pl.ds(s * RPT + t * K, K)],
                        out.at[c, pl.ds(s * RPT + t * K, K)])


def _norm(deg):
    return lax.rsqrt(jnp.maximum(deg, 1.0))


def _scale_body(x_ref, do0_ref, do1_ref, oa_ref, ob_ref):
    scaled = x_ref[...] * _norm(do0_ref[...] + do1_ref[...])
    oa_ref[...] = scaled[:, :DH]
    ob_ref[...] = scaled[:, DH:]


def _mid_body(pa_ref, pb_ref, di0_ref, di1_ref,
              do0_ref, do1_ref, w_ref, oa_ref, ob_ref):
    agg = jnp.concatenate([pa_ref[...], pb_ref[...]], axis=1)
    agg = agg * _norm(di0_ref[...] + di1_ref[...])
    hidden = jnp.dot(agg, w_ref[...], preferred_element_type=jnp.float32)
    scaled = jnp.maximum(hidden, 0.0) * _norm(do0_ref[...] + do1_ref[...])
    oa_ref[...] = scaled[:, :DH]
    ob_ref[...] = scaled[:, DH:]


def _final_body(pa_ref, pb_ref, di0_ref, di1_ref, w_ref, o_ref):
    agg = jnp.concatenate([pa_ref[...], pb_ref[...]], axis=1)
    agg = agg * _norm(di0_ref[...] + di1_ref[...])
    o_ref[...] = jnp.dot(agg, w_ref[...], preferred_element_type=jnp.float32)


_row_spec = pl.BlockSpec((BN, D), lambda i: (i, 0))
_half_spec = pl.BlockSpec((BN, DH), lambda i: (i, 0))
_col_spec = pl.BlockSpec((BN, 1), lambda i: (i, 0))
_w_spec = pl.BlockSpec((D, D), lambda i: (0, 0))
_out_struct = jax.ShapeDtypeStruct((NP, D), jnp.float32)
_half_struct = jax.ShapeDtypeStruct((NP, DH), jnp.float32)

_scale_call = pl.pallas_call(
    _scale_body, grid=(GB,),
    in_specs=[_row_spec, _col_spec, _col_spec],
    out_specs=(_half_spec, _half_spec),
    out_shape=(_half_struct, _half_struct))

_mid_call = pl.pallas_call(
    _mid_body, grid=(GB,),
    in_specs=[_half_spec, _half_spec,
              _col_spec, _col_spec, _col_spec, _col_spec, _w_spec],
    out_specs=(_half_spec, _half_spec),
    out_shape=(_half_struct, _half_struct))

_final_call = pl.pallas_call(
    _final_body, grid=(GB,),
    in_specs=[_half_spec, _half_spec, _col_spec, _col_spec, _w_spec],
    out_specs=_row_spec, out_shape=_out_struct)


def kernel(features, edge_index, W1, W2):
    src = edge_index[0].astype(jnp.int32)
    dst = edge_index[1].astype(jnp.int32)
    pad = jnp.full((EP - E,), N, jnp.int32)
    srcp = jnp.concatenate([src, pad]).reshape(EROWS, K)
    dstp = jnp.concatenate([dst, pad]).reshape(EROWS, K)
    xpad = jnp.pad(features, ((0, NP - N), (0, 0)))

    deg = _deg_kernel(srcp, dstp)
    degf = deg.reshape(NC, 2, NP).astype(jnp.float32)
    do0 = degf[0, 0].reshape(NP, 1)
    do1 = degf[1, 0].reshape(NP, 1)
    di0 = degf[0, 1].reshape(NP, 1)
    di1 = degf[1, 1].reshape(NP, 1)

    ha, hb = _scale_call(xpad, do0, do1)
    p = _mp_kernel(ha, hb, srcp, dstp)
    ha2, hb2 = _mid_call(p[0], p[1], di0, di1, do0, do1, W1)
    p2 = _mp_kernel(ha2, hb2, srcp, dstp)
    out = _final_call(p2[0], p2[1], di0, di1, W2)
    return out[:N]
